# Initial kernel scaffold; baseline (speedup 1.0000x reference)
import jax
import jax.numpy as jnp
from jax.experimental import pallas as pl


def kernel(xyz, feat, W1, g1, b1, W2, g2, b2):
    raise NotImplementedError("write your pallas kernel here")


# speedup vs baseline: 13.5814x; 13.5814x over previous
import functools

import jax
import jax.numpy as jnp
from jax import lax
from jax.experimental import pallas as pl
from jax.experimental.pallas import tpu as pltpu
from jax.experimental.pallas import tpu_sc as plsc

B = 8
N = 4096
S = 1024
KNN = 16
CIN = 128
COUT = 256
ROWS = B * S * KNN

_BIG = 3.0e38


def _fps_body(init_ref, x_ref, cent_ref, cx_ref, cy_ref, cz_ref):
    X = x_ref[:, 0, :]
    Y = x_ref[:, 1, :]
    Z = x_ref[:, 2, :]
    lanes = lax.broadcasted_iota(jnp.int32, (B, N), 1)

    far0 = init_ref[:, 0:1]
    l128 = lax.broadcasted_iota(jnp.int32, (B, 128), 1)

    def body(j, carry):
        dist, far, bc, bx, by, bz = carry
        onehot = lanes == far
        cx = jnp.sum(jnp.where(onehot, X, 0.0), axis=1, keepdims=True)
        cy = jnp.sum(jnp.where(onehot, Y, 0.0), axis=1, keepdims=True)
        cz = jnp.sum(jnp.where(onehot, Z, 0.0), axis=1, keepdims=True)
        sel = l128 == j
        bc = jnp.where(sel, far, bc)
        bx = jnp.where(sel, cx, bx)
        by = jnp.where(sel, cy, by)
        bz = jnp.where(sel, cz, bz)
        d = (X - cx) ** 2 + (Y - cy) ** 2 + (Z - cz) ** 2
        dist = jnp.minimum(dist, d)
        m = jnp.max(dist, axis=1, keepdims=True)
        far_new = jnp.min(jnp.where(dist == m, lanes, N), axis=1, keepdims=True)
        return dist, far_new.astype(jnp.int32), bc, bx, by, bz

    dist = jnp.full((B, N), 1.0e10, dtype=jnp.float32)
    far = far0
    zi = jnp.zeros((B, 128), jnp.int32)
    zf = jnp.zeros((B, 128), jnp.float32)
    for o in range(S // 128):
        dist, far, bc, bx, by, bz = lax.fori_loop(
            0, 128, body, (dist, far, zi, zf, zf, zf))
        cent_ref[:, o * 128:(o + 1) * 128] = bc
        cx_ref[:, o * 128:(o + 1) * 128] = bx
        cy_ref[:, o * 128:(o + 1) * 128] = by
        cz_ref[:, o * 128:(o + 1) * 128] = bz


def _fps_call(xyz3, init):
    return pl.pallas_call(
        _fps_body,
        out_shape=(
            jax.ShapeDtypeStruct((B, S), jnp.int32),
            jax.ShapeDtypeStruct((B, S), jnp.float32),
            jax.ShapeDtypeStruct((B, S), jnp.float32),
            jax.ShapeDtypeStruct((B, S), jnp.float32),
        ),
    )(init, xyz3)


_QT = 256


def _knn_body(x_ref, q_ref, idx_ref):
    X = x_ref[0, 0:1, :]
    Y = x_ref[0, 1:2, :]
    Z = x_ref[0, 2:3, :]
    qx = q_ref[0, :, 0:1]
    qy = q_ref[0, :, 1:2]
    qz = q_ref[0, :, 2:3]
    vals = (qx - X) ** 2 + (qy - Y) ** 2 + (qz - Z) ** 2
    lanes = lax.broadcasted_iota(jnp.int32, (_QT, N), 1)
    l16 = lax.broadcasted_iota(jnp.int32, (_QT, KNN), 1)
    out = jnp.zeros((_QT, KNN), dtype=jnp.int32)
    for k in range(KNN):
        m = jnp.min(vals, axis=1, keepdims=True)
        am = jnp.min(jnp.where(vals == m, lanes, N), axis=1, keepdims=True)
        out = jnp.where(l16 == k, am, out)
        vals = jnp.where(lanes == am, _BIG, vals)
    idx_ref[0] = out


def _knn_call(xyz3, new_xyz):
    return pl.pallas_call(
        _knn_body,
        grid=(B, S // _QT),
        in_specs=[
            pl.BlockSpec((1, 3, N), lambda b, s: (b, 0, 0)),
            pl.BlockSpec((1, _QT, 3), lambda b, s: (b, s, 0)),
        ],
        out_specs=pl.BlockSpec((1, _QT, KNN), lambda b, s: (b, s, 0)),
        out_shape=jax.ShapeDtypeStruct((B, S, KNN), jnp.int32),
    )(xyz3, new_xyz)


_NW = 32
_RPW = ROWS // _NW
_CH = 128
_NCH = _RPW // _CH


def _gather_body(feat_hbm, xt_hbm, yt_hbm, zt_hbm, idx_hbm, gfeat_hbm, gxyzf_hbm,
                 idx_v, frows_v, xrows_v, xt_v, yt_v, zt_v, sem1):
    wid = lax.axis_index("s") * 2 + lax.axis_index("c")
    base = wid * _RPW

    pltpu.sync_copy(xt_hbm, xt_v)
    pltpu.sync_copy(yt_hbm, yt_v)
    pltpu.sync_copy(zt_hbm, zt_v)

    zeros16 = jnp.zeros((16,), jnp.float32)

    def zinit(j, carry):
        xrows_v[pl.ds(j * 16, 16)] = zeros16
        return carry

    lax.fori_loop(0, _CH * 16 // 16, zinit, 0)

    def chunk(c, carry):
        off = base + c * _CH
        pltpu.sync_copy(idx_hbm.at[pl.ds(off, _CH)], idx_v)
        cp1 = pltpu.async_copy(feat_hbm.at[idx_v], frows_v, sem1)
        for j in range(_CH // 16):
            iv = idx_v[pl.ds(j * 16, 16)]
            pos = (lax.iota(jnp.int32, 16) + j * 16) * 16
            for ci, tv in ((0, xt_v), (1, yt_v), (2, zt_v)):
                vals = plsc.load_gather(tv, [iv])
                plsc.store_scatter(xrows_v, [pos + ci], vals)
        cp1.wait()
        pltpu.sync_copy(frows_v, gfeat_hbm.at[pl.ds(off, _CH)])
        pltpu.sync_copy(xrows_v, gxyzf_hbm.at[pl.ds(off * 16, _CH * 16)])
        return carry

    lax.fori_loop(0, _NCH, chunk, 0)


@functools.lru_cache(maxsize=None)
def _make_gather():
    return pl.kernel(
        _gather_body,
        out_type=(
            jax.ShapeDtypeStruct((ROWS, CIN), jnp.float32),
            jax.ShapeDtypeStruct((ROWS * 16,), jnp.float32),
        ),
        mesh=plsc.VectorSubcoreMesh(core_axis_name="c", subcore_axis_name="s"),
        compiler_params=pltpu.CompilerParams(needs_layout_passes=False),
        scratch_types=[
            pltpu.VMEM((_CH,), jnp.int32),
            pltpu.VMEM((_CH, CIN), jnp.float32),
            pltpu.VMEM((_CH * 16,), jnp.float32),
            pltpu.VMEM((B * N,), jnp.float32),
            pltpu.VMEM((B * N,), jnp.float32),
            pltpu.VMEM((B * N,), jnp.float32),
            pltpu.SemaphoreType.DMA,
        ],
    )


def _gather_call(feat_flat, xt, yt, zt, idx_flat):
    gfeat, gxyzf = _make_gather()(feat_flat, xt, yt, zt, idx_flat)
    return gfeat, gxyzf.reshape(ROWS, 16)


_RT = 2048


def _matmul_ext(gf, gx, qx, w_ref):
    y = jnp.dot(gf, w_ref[0:CIN, :], preferred_element_type=jnp.float32)
    y += jnp.dot(gx, w_ref[CIN:CIN + 16, :], preferred_element_type=jnp.float32)
    y += jnp.dot(qx, w_ref[CIN + 16:CIN + 32, :], preferred_element_type=jnp.float32)
    return y


def _mlp1_body(gf_ref, gx_ref, qx_ref, w_ref, s_ref, ss_ref):
    i = pl.program_id(0)
    y = _matmul_ext(gf_ref[...], gx_ref[...], qx_ref[...], w_ref)

    @pl.when(i == 0)
    def _():
        s_ref[...] = jnp.zeros_like(s_ref)
        ss_ref[...] = jnp.zeros_like(ss_ref)

    s_ref[...] += jnp.sum(y, axis=0, keepdims=True)
    ss_ref[...] += jnp.sum(y * y, axis=0, keepdims=True)


def _mlp1_call(gfeat, gxyz, qexp, wext):
    return pl.pallas_call(
        _mlp1_body,
        grid=(ROWS // _RT,),
        in_specs=[
            pl.BlockSpec((_RT, CIN), lambda i: (i, 0)),
            pl.BlockSpec((_RT, 16), lambda i: (i, 0)),
            pl.BlockSpec((_RT, 16), lambda i: (i, 0)),
            pl.BlockSpec((CIN + 32, COUT), lambda i: (0, 0)),
        ],
        out_specs=(
            pl.BlockSpec((1, COUT), lambda i: (0, 0)),
            pl.BlockSpec((1, COUT), lambda i: (0, 0)),
        ),
        out_shape=(
            jax.ShapeDtypeStruct((1, COUT), jnp.float32),
            jax.ShapeDtypeStruct((1, COUT), jnp.float32),
        ),
    )(gfeat, gxyz, qexp, wext)


def _mlp2_body(gf_ref, gx_ref, qx_ref, w_ref, w2_ref, sc_ref, sh_ref,
               y2_ref, s_ref, ss_ref):
    i = pl.program_id(0)
    y = _matmul_ext(gf_ref[...], gx_ref[...], qx_ref[...], w_ref)
    h = jnp.maximum(y * sc_ref[...] + sh_ref[...], 0.0)
    y2 = jnp.dot(h, w2_ref[...], preferred_element_type=jnp.float32)
    y2_ref[...] = y2

    @pl.when(i == 0)
    def _():
        s_ref[...] = jnp.zeros_like(s_ref)
        ss_ref[...] = jnp.zeros_like(ss_ref)

    s_ref[...] += jnp.sum(y2, axis=0, keepdims=True)
    ss_ref[...] += jnp.sum(y2 * y2, axis=0, keepdims=True)


def _mlp2_call(gfeat, gxyz, qexp, wext, w2t, sc1, sh1):
    return pl.pallas_call(
        _mlp2_body,
        grid=(ROWS // _RT,),
        in_specs=[
            pl.BlockSpec((_RT, CIN), lambda i: (i, 0)),
            pl.BlockSpec((_RT, 16), lambda i: (i, 0)),
            pl.BlockSpec((_RT, 16), lambda i: (i, 0)),
            pl.BlockSpec((CIN + 32, COUT), lambda i: (0, 0)),
            pl.BlockSpec((COUT, COUT), lambda i: (0, 0)),
            pl.BlockSpec((1, COUT), lambda i: (0, 0)),
            pl.BlockSpec((1, COUT), lambda i: (0, 0)),
        ],
        out_specs=(
            pl.BlockSpec((_RT, COUT), lambda i: (i, 0)),
            pl.BlockSpec((1, COUT), lambda i: (0, 0)),
            pl.BlockSpec((1, COUT), lambda i: (0, 0)),
        ),
        out_shape=(
            jax.ShapeDtypeStruct((ROWS, COUT), jnp.float32),
            jax.ShapeDtypeStruct((1, COUT), jnp.float32),
            jax.ShapeDtypeStruct((1, COUT), jnp.float32),
        ),
    )(gfeat, gxyz, qexp, wext, w2t, sc1, sh1)


_QT3 = 512


def _mlp3_body(y_ref, sc_ref, sh_ref, o_ref):
    k = pl.program_id(1)
    z = jnp.maximum(y_ref[0] * sc_ref[...] + sh_ref[...], 0.0)

    @pl.when(k == 0)
    def _():
        o_ref[...] = z

    @pl.when(k > 0)
    def _():
        o_ref[...] = jnp.maximum(o_ref[...], z)


def _mlp3_call(y2k, sc2, sh2):
    return pl.pallas_call(
        _mlp3_body,
        grid=(B * S // _QT3, KNN),
        in_specs=[
            pl.BlockSpec((1, _QT3, COUT), lambda q, k: (k, q, 0)),
            pl.BlockSpec((1, COUT), lambda q, k: (0, 0)),
            pl.BlockSpec((1, COUT), lambda q, k: (0, 0)),
        ],
        out_specs=pl.BlockSpec((_QT3, COUT), lambda q, k: (q, 0)),
        out_shape=jax.ShapeDtypeStruct((B * S, COUT), jnp.float32),
    )(y2k, sc2, sh2)


def kernel(xyz, feat, W1, g1, b1, W2, g2, b2):
    xyz3 = jnp.transpose(xyz, (0, 2, 1))
    far0 = jax.random.randint(jax.random.key(1), (B,), 0, N, dtype=jnp.int32)
    init = jnp.broadcast_to(far0[:, None], (B, 128))

    cent, cxs, cys, czs = _fps_call(xyz3, init)
    new_xyz = jnp.stack([cxs, cys, czs], axis=-1)

    idx = _knn_call(xyz3, new_xyz)

    idx_km = jnp.transpose(idx, (2, 0, 1))
    idx_flat = (idx_km + (jnp.arange(B, dtype=jnp.int32) * N)[None, :, None])
    idx_flat = idx_flat.reshape(ROWS).astype(jnp.int32)

    feat_flat = feat.reshape(B * N, CIN)
    xt = xyz[:, :, 0].reshape(B * N)
    yt = xyz[:, :, 1].reshape(B * N)
    zt = xyz[:, :, 2].reshape(B * N)
    gfeat, gxyz = _gather_call(feat_flat, xt, yt, zt, idx_flat)

    nq = jnp.pad(new_xyz, ((0, 0), (0, 0), (0, 13))).reshape(1, B * S, 16)
    qexp = jnp.broadcast_to(nq, (KNN, B * S, 16)).reshape(ROWS, 16)

    w1t = W1.T
    zpad = jnp.zeros((13, COUT), dtype=jnp.float32)
    wext = jnp.concatenate(
        [w1t[:CIN], w1t[CIN:], zpad, -w1t[CIN:], zpad], axis=0)

    s1, ss1 = _mlp1_call(gfeat, gxyz, qexp, wext)
    n = jnp.float32(ROWS)
    m1 = s1 / n
    v1 = ss1 / n - m1 * m1
    sc1 = g1[None, :] * lax.rsqrt(v1 + 1e-5)
    sh1 = b1[None, :] - m1 * sc1

    y2, s2, ss2 = _mlp2_call(gfeat, gxyz, qexp, wext, W2.T, sc1, sh1)
    m2 = s2 / n
    v2 = ss2 / n - m2 * m2
    sc2 = g2[None, :] * lax.rsqrt(v2 + 1e-5)
    sh2 = b2[None, :] - m2 * sc2

    nf = _mlp3_call(y2.reshape(KNN, B * S, COUT), sc2, sh2)
    return new_xyz, nf.reshape(B, S, COUT)


# speedup vs baseline: 16.4049x; 1.2079x over previous
import functools

import jax
import jax.numpy as jnp
from jax import lax
from jax.experimental import pallas as pl
from jax.experimental.pallas import tpu as pltpu
from jax.experimental.pallas import tpu_sc as plsc

B = 8
N = 4096
S = 1024
KNN = 16
CIN = 128
COUT = 256
ROWS = B * S * KNN

_BIG = 3.0e38


def _fps_body(init_ref, x_ref, cx_ref, cy_ref, cz_ref):
    X = x_ref[:, 0, :]
    Y = x_ref[:, 1, :]
    Z = x_ref[:, 2, :]
    lanes = lax.broadcasted_iota(jnp.int32, (B, N), 1)

    far0 = init_ref[:, 0:1]
    l128 = lax.broadcasted_iota(jnp.int32, (B, 128), 1)

    oh0 = (lanes == far0).astype(jnp.float32)

    def body(j, carry):
        dist, oh, bx, by, bz = carry
        cx = jnp.sum(oh * X, axis=1, keepdims=True)
        cy = jnp.sum(oh * Y, axis=1, keepdims=True)
        cz = jnp.sum(oh * Z, axis=1, keepdims=True)
        sel = l128 == j
        bx = jnp.where(sel, cx, bx)
        by = jnp.where(sel, cy, by)
        bz = jnp.where(sel, cz, bz)
        d = (X - cx) ** 2 + (Y - cy) ** 2 + (Z - cz) ** 2
        dist = jnp.minimum(dist, d)
        m = jnp.max(dist, axis=1, keepdims=True)
        oh = (dist == m).astype(jnp.float32)
        return dist, oh, bx, by, bz

    dist = jnp.full((B, N), 1.0e10, dtype=jnp.float32)
    oh = oh0
    zf = jnp.zeros((B, 128), jnp.float32)
    for o in range(S // 128):
        dist, oh, bx, by, bz = lax.fori_loop(
            0, 128, body, (dist, oh, zf, zf, zf))
        cx_ref[:, o * 128:(o + 1) * 128] = bx
        cy_ref[:, o * 128:(o + 1) * 128] = by
        cz_ref[:, o * 128:(o + 1) * 128] = bz


def _fps_call(xyz3, init):
    return pl.pallas_call(
        _fps_body,
        out_shape=(
            jax.ShapeDtypeStruct((B, S), jnp.float32),
            jax.ShapeDtypeStruct((B, S), jnp.float32),
            jax.ShapeDtypeStruct((B, S), jnp.float32),
        ),
    )(init, xyz3)


_QT = 256


def _knn_body(x_ref, q_ref, idx_ref):
    X = x_ref[0, 0:1, :]
    Y = x_ref[0, 1:2, :]
    Z = x_ref[0, 2:3, :]
    qx = q_ref[0, :, 0:1]
    qy = q_ref[0, :, 1:2]
    qz = q_ref[0, :, 2:3]
    vals = (qx - X) ** 2 + (qy - Y) ** 2 + (qz - Z) ** 2
    lanes = lax.broadcasted_iota(jnp.int32, (_QT, N), 1)
    l16 = lax.broadcasted_iota(jnp.int32, (_QT, KNN), 1)
    out = jnp.zeros((_QT, KNN), dtype=jnp.int32)
    for k in range(KNN):
        m = jnp.min(vals, axis=1, keepdims=True)
        am = jnp.min(jnp.where(vals == m, lanes, N), axis=1, keepdims=True)
        out = jnp.where(l16 == k, am, out)
        vals = jnp.where(lanes == am, _BIG, vals)
    idx_ref[0] = out


def _knn_call(xyz3, new_xyz):
    return pl.pallas_call(
        _knn_body,
        grid=(B, S // _QT),
        in_specs=[
            pl.BlockSpec((1, 3, N), lambda b, s: (b, 0, 0)),
            pl.BlockSpec((1, _QT, 3), lambda b, s: (b, s, 0)),
        ],
        out_specs=pl.BlockSpec((1, _QT, KNN), lambda b, s: (b, s, 0)),
        out_shape=jax.ShapeDtypeStruct((B, S, KNN), jnp.int32),
    )(xyz3, new_xyz)


_NW = 32
_RPW = ROWS // _NW
_CH = 128
_NCH = _RPW // _CH


def _gather_body(feat_hbm, xt_hbm, yt_hbm, zt_hbm, idx_hbm, gfeat_hbm, gxyzf_hbm,
                 idx_v, frows_v, xrows_v, xt_v, yt_v, zt_v, sem1):
    wid = lax.axis_index("s") * 2 + lax.axis_index("c")
    base = wid * _RPW

    pltpu.sync_copy(xt_hbm, xt_v)
    pltpu.sync_copy(yt_hbm, yt_v)
    pltpu.sync_copy(zt_hbm, zt_v)

    zeros16 = jnp.zeros((16,), jnp.float32)

    def zinit(j, carry):
        xrows_v[pl.ds(j * 16, 16)] = zeros16
        return carry

    lax.fori_loop(0, _CH * 16 // 16, zinit, 0)

    def chunk(c, carry):
        off = base + c * _CH
        pltpu.sync_copy(idx_hbm.at[pl.ds(off, _CH)], idx_v)
        cp1 = pltpu.async_copy(feat_hbm.at[idx_v], frows_v, sem1)
        for j in range(_CH // 16):
            iv = idx_v[pl.ds(j * 16, 16)]
            pos = (lax.iota(jnp.int32, 16) + j * 16) * 16
            for ci, tv in ((0, xt_v), (1, yt_v), (2, zt_v)):
                vals = plsc.load_gather(tv, [iv])
                plsc.store_scatter(xrows_v, [pos + ci], vals)
        cp1.wait()
        pltpu.sync_copy(frows_v, gfeat_hbm.at[pl.ds(off, _CH)])
        pltpu.sync_copy(xrows_v, gxyzf_hbm.at[pl.ds(off * 16, _CH * 16)])
        return carry

    lax.fori_loop(0, _NCH, chunk, 0)


@functools.lru_cache(maxsize=None)
def _make_gather():
    return pl.kernel(
        _gather_body,
        out_type=(
            jax.ShapeDtypeStruct((ROWS, CIN), jnp.float32),
            jax.ShapeDtypeStruct((ROWS * 16,), jnp.float32),
        ),
        mesh=plsc.VectorSubcoreMesh(core_axis_name="c", subcore_axis_name="s"),
        compiler_params=pltpu.CompilerParams(needs_layout_passes=False),
        scratch_types=[
            pltpu.VMEM((_CH,), jnp.int32),
            pltpu.VMEM((_CH, CIN), jnp.float32),
            pltpu.VMEM((_CH * 16,), jnp.float32),
            pltpu.VMEM((B * N,), jnp.float32),
            pltpu.VMEM((B * N,), jnp.float32),
            pltpu.VMEM((B * N,), jnp.float32),
            pltpu.SemaphoreType.DMA,
        ],
    )


def _gather_call(feat_flat, xt, yt, zt, idx_flat):
    gfeat, gxyzf = _make_gather()(feat_flat, xt, yt, zt, idx_flat)
    return gfeat, gxyzf.reshape(ROWS, 16)


_RT = 2048


def _matmul_ext(gf, gx, qx, w_ref):
    y = jnp.dot(gf, w_ref[0:CIN, :], preferred_element_type=jnp.float32)
    y += jnp.dot(gx, w_ref[CIN:CIN + 16, :], preferred_element_type=jnp.float32)
    y += jnp.dot(qx, w_ref[CIN + 16:CIN + 32, :], preferred_element_type=jnp.float32)
    return y


def _mlp1_body(gf_ref, gx_ref, qx_ref, w_ref, s_ref, ss_ref):
    i = pl.program_id(0)
    y = _matmul_ext(gf_ref[...], gx_ref[...], qx_ref[...], w_ref)

    @pl.when(i == 0)
    def _():
        s_ref[...] = jnp.zeros_like(s_ref)
        ss_ref[...] = jnp.zeros_like(ss_ref)

    s_ref[...] += jnp.sum(y, axis=0, keepdims=True)
    ss_ref[...] += jnp.sum(y * y, axis=0, keepdims=True)


_GT = B * S // _RT


def _mlp1_call(gfeat, gxyz, qexp, wext):
    return pl.pallas_call(
        _mlp1_body,
        grid=(ROWS // _RT,),
        in_specs=[
            pl.BlockSpec((_RT, CIN), lambda i: (i, 0)),
            pl.BlockSpec((_RT, 16), lambda i: (i, 0)),
            pl.BlockSpec((_RT, 16), lambda i: (i % _GT, 0)),
            pl.BlockSpec((CIN + 32, COUT), lambda i: (0, 0)),
        ],
        out_specs=(
            pl.BlockSpec((1, COUT), lambda i: (0, 0)),
            pl.BlockSpec((1, COUT), lambda i: (0, 0)),
        ),
        out_shape=(
            jax.ShapeDtypeStruct((1, COUT), jnp.float32),
            jax.ShapeDtypeStruct((1, COUT), jnp.float32),
        ),
    )(gfeat, gxyz, qexp, wext)


def _mlp2_body(gf_ref, gx_ref, qx_ref, w_ref, w2_ref, sc_ref, sh_ref,
               s_ref, ss_ref):
    i = pl.program_id(0)
    y = _matmul_ext(gf_ref[...], gx_ref[...], qx_ref[...], w_ref)
    h = jnp.maximum(y * sc_ref[...] + sh_ref[...], 0.0)
    y2 = jnp.dot(h, w2_ref[...], preferred_element_type=jnp.float32)

    @pl.when(i == 0)
    def _():
        s_ref[...] = jnp.zeros_like(s_ref)
        ss_ref[...] = jnp.zeros_like(ss_ref)

    s_ref[...] += jnp.sum(y2, axis=0, keepdims=True)
    ss_ref[...] += jnp.sum(y2 * y2, axis=0, keepdims=True)


def _mlp2_call(gfeat, gxyz, qexp, wext, w2t, sc1, sh1):
    return pl.pallas_call(
        _mlp2_body,
        grid=(ROWS // _RT,),
        in_specs=[
            pl.BlockSpec((_RT, CIN), lambda i: (i, 0)),
            pl.BlockSpec((_RT, 16), lambda i: (i, 0)),
            pl.BlockSpec((_RT, 16), lambda i: (i % _GT, 0)),
            pl.BlockSpec((CIN + 32, COUT), lambda i: (0, 0)),
            pl.BlockSpec((COUT, COUT), lambda i: (0, 0)),
            pl.BlockSpec((1, COUT), lambda i: (0, 0)),
            pl.BlockSpec((1, COUT), lambda i: (0, 0)),
        ],
        out_specs=(
            pl.BlockSpec((1, COUT), lambda i: (0, 0)),
            pl.BlockSpec((1, COUT), lambda i: (0, 0)),
        ),
        out_shape=(
            jax.ShapeDtypeStruct((1, COUT), jnp.float32),
            jax.ShapeDtypeStruct((1, COUT), jnp.float32),
        ),
    )(gfeat, gxyz, qexp, wext, w2t, sc1, sh1)


def _mlp3_body(gf_ref, gx_ref, qx_ref, w_ref, w2_ref, sc1_ref, sh1_ref,
               sc2_ref, sh2_ref, o_ref):
    k = pl.program_id(1)
    y = _matmul_ext(gf_ref[...], gx_ref[...], qx_ref[...], w_ref)
    h = jnp.maximum(y * sc1_ref[...] + sh1_ref[...], 0.0)
    y2 = jnp.dot(h, w2_ref[...], preferred_element_type=jnp.float32)
    z = jnp.maximum(y2 * sc2_ref[...] + sh2_ref[...], 0.0)

    @pl.when(k == 0)
    def _():
        o_ref[...] = z

    @pl.when(k > 0)
    def _():
        o_ref[...] = jnp.maximum(o_ref[...], z)


def _mlp3_call(gfeat, gxyz, qexp, wext, w2t, sc1, sh1, sc2, sh2):
    return pl.pallas_call(
        _mlp3_body,
        grid=(_GT, KNN),
        in_specs=[
            pl.BlockSpec((_RT, CIN), lambda g, k: (k * _GT + g, 0)),
            pl.BlockSpec((_RT, 16), lambda g, k: (k * _GT + g, 0)),
            pl.BlockSpec((_RT, 16), lambda g, k: (g, 0)),
            pl.BlockSpec((CIN + 32, COUT), lambda g, k: (0, 0)),
            pl.BlockSpec((COUT, COUT), lambda g, k: (0, 0)),
            pl.BlockSpec((1, COUT), lambda g, k: (0, 0)),
            pl.BlockSpec((1, COUT), lambda g, k: (0, 0)),
            pl.BlockSpec((1, COUT), lambda g, k: (0, 0)),
            pl.BlockSpec((1, COUT), lambda g, k: (0, 0)),
        ],
        out_specs=pl.BlockSpec((_RT, COUT), lambda g, k: (g, 0)),
        out_shape=jax.ShapeDtypeStruct((B * S, COUT), jnp.float32),
    )(gfeat, gxyz, qexp, wext, w2t, sc1, sh1, sc2, sh2)


def kernel(xyz, feat, W1, g1, b1, W2, g2, b2):
    xyz3 = jnp.transpose(xyz, (0, 2, 1))
    far0 = jax.random.randint(jax.random.key(1), (B,), 0, N, dtype=jnp.int32)
    init = jnp.broadcast_to(far0[:, None], (B, 128))

    cxs, cys, czs = _fps_call(xyz3, init)
    new_xyz = jnp.stack([cxs, cys, czs], axis=-1)

    idx = _knn_call(xyz3, new_xyz)

    idx_km = jnp.transpose(idx, (2, 0, 1))
    idx_flat = (idx_km + (jnp.arange(B, dtype=jnp.int32) * N)[None, :, None])
    idx_flat = idx_flat.reshape(ROWS).astype(jnp.int32)

    feat_flat = feat.reshape(B * N, CIN)
    xt = xyz[:, :, 0].reshape(B * N)
    yt = xyz[:, :, 1].reshape(B * N)
    zt = xyz[:, :, 2].reshape(B * N)
    gfeat, gxyz = _gather_call(feat_flat, xt, yt, zt, idx_flat)

    qexp = jnp.pad(new_xyz, ((0, 0), (0, 0), (0, 13))).reshape(B * S, 16)

    w1t = W1.T
    zpad = jnp.zeros((13, COUT), dtype=jnp.float32)
    wext = jnp.concatenate(
        [w1t[:CIN], w1t[CIN:], zpad, -w1t[CIN:], zpad], axis=0)

    s1, ss1 = _mlp1_call(gfeat, gxyz, qexp, wext)
    n = jnp.float32(ROWS)
    m1 = s1 / n
    v1 = ss1 / n - m1 * m1
    sc1 = g1[None, :] * lax.rsqrt(v1 + 1e-5)
    sh1 = b1[None, :] - m1 * sc1

    s2, ss2 = _mlp2_call(gfeat, gxyz, qexp, wext, W2.T, sc1, sh1)
    m2 = s2 / n
    v2 = ss2 / n - m2 * m2
    sc2 = g2[None, :] * lax.rsqrt(v2 + 1e-5)
    sh2 = b2[None, :] - m2 * sc2

    nf = _mlp3_call(gfeat, gxyz, qexp, wext, W2.T, sc1, sh1, sc2, sh2)
    return new_xyz, nf.reshape(B, S, COUT)


# speedup vs baseline: 17.7429x; 1.0816x over previous
import functools

import jax
import jax.numpy as jnp
from jax import lax
from jax.experimental import pallas as pl
from jax.experimental.pallas import tpu as pltpu
from jax.experimental.pallas import tpu_sc as plsc

B = 8
N = 4096
S = 1024
KNN = 16
CIN = 128
COUT = 256
ROWS = B * S * KNN

_BIG = 3.0e38


def _fps_body(init_ref, x_ref, cx_ref, cy_ref, cz_ref):
    X = x_ref[:, 0, :]
    Y = x_ref[:, 1, :]
    Z = x_ref[:, 2, :]
    lanes = lax.broadcasted_iota(jnp.int32, (B, N), 1)

    far0 = init_ref[:, 0:1]
    l128 = lax.broadcasted_iota(jnp.int32, (B, 128), 1)

    oh0 = (lanes == far0).astype(jnp.float32)

    def body(j, carry):
        dist, oh, bx, by, bz = carry
        cx = jnp.sum(oh * X, axis=1, keepdims=True)
        cy = jnp.sum(oh * Y, axis=1, keepdims=True)
        cz = jnp.sum(oh * Z, axis=1, keepdims=True)
        sel = l128 == j
        bx = jnp.where(sel, cx, bx)
        by = jnp.where(sel, cy, by)
        bz = jnp.where(sel, cz, bz)
        d = (X - cx) ** 2 + (Y - cy) ** 2 + (Z - cz) ** 2
        dist = jnp.minimum(dist, d)
        m = jnp.max(dist, axis=1, keepdims=True)
        oh = (dist == m).astype(jnp.float32)
        return dist, oh, bx, by, bz

    dist = jnp.full((B, N), 1.0e10, dtype=jnp.float32)
    oh = oh0
    zf = jnp.zeros((B, 128), jnp.float32)
    for o in range(S // 128):
        dist, oh, bx, by, bz = lax.fori_loop(
            0, 128, body, (dist, oh, zf, zf, zf))
        cx_ref[:, o * 128:(o + 1) * 128] = bx
        cy_ref[:, o * 128:(o + 1) * 128] = by
        cz_ref[:, o * 128:(o + 1) * 128] = bz


def _fps_call(xyz3, init):
    return pl.pallas_call(
        _fps_body,
        out_shape=(
            jax.ShapeDtypeStruct((B, S), jnp.float32),
            jax.ShapeDtypeStruct((B, S), jnp.float32),
            jax.ShapeDtypeStruct((B, S), jnp.float32),
        ),
    )(init, xyz3)


_QT = 256


def _knn_body(x_ref, q_ref, idx_ref):
    X = x_ref[0, 0:1, :]
    Y = x_ref[0, 1:2, :]
    Z = x_ref[0, 2:3, :]
    qx = q_ref[0, :, 0:1]
    qy = q_ref[0, :, 1:2]
    qz = q_ref[0, :, 2:3]
    vals = (qx - X) ** 2 + (qy - Y) ** 2 + (qz - Z) ** 2
    lanes = lax.broadcasted_iota(jnp.int32, (_QT, N), 1)
    l16 = lax.broadcasted_iota(jnp.int32, (_QT, KNN), 1)
    out = jnp.zeros((_QT, KNN), dtype=jnp.int32)
    for k in range(KNN):
        m = jnp.min(vals, axis=1, keepdims=True)
        eqm = vals == m
        am = jnp.min(jnp.where(eqm, lanes, N), axis=1, keepdims=True)
        out = jnp.where(l16 == k, am, out)
        vals = jnp.where(eqm, _BIG, vals)
    idx_ref[0] = out


def _knn_call(xyz3, new_xyz):
    return pl.pallas_call(
        _knn_body,
        grid=(B, S // _QT),
        in_specs=[
            pl.BlockSpec((1, 3, N), lambda b, s: (b, 0, 0)),
            pl.BlockSpec((1, _QT, 3), lambda b, s: (b, s, 0)),
        ],
        out_specs=pl.BlockSpec((1, _QT, KNN), lambda b, s: (b, s, 0)),
        out_shape=jax.ShapeDtypeStruct((B, S, KNN), jnp.int32),
    )(xyz3, new_xyz)


_NW = 32
_RPW = ROWS // _NW
_CH = 128
_NCH = _RPW // _CH


def _gather_body(feat_hbm, xt_hbm, yt_hbm, zt_hbm, idx_hbm, gfeat_hbm, gxyzf_hbm,
                 idx_a, idx_b, frows_a, frows_b, xrows_v, xt_v, yt_v, zt_v,
                 sem_a, sem_b):
    wid = lax.axis_index("s") * 2 + lax.axis_index("c")
    base = wid * _RPW
    half = (wid % 2) * (4 * N)
    pltpu.sync_copy(xt_hbm.at[pl.ds(half, 4 * N)], xt_v)
    pltpu.sync_copy(yt_hbm.at[pl.ds(half, 4 * N)], yt_v)
    pltpu.sync_copy(zt_hbm.at[pl.ds(half, 4 * N)], zt_v)

    zeros16 = jnp.zeros((16,), jnp.float32)

    def zinit(j, carry):
        xrows_v[pl.ds(j * 16, 16)] = zeros16
        return carry

    lax.fori_loop(0, _CH * 16 // 16, zinit, 0)

    def xyz_block(idxbuf, off):
        for j in range(_CH // 16):
            iv = idxbuf[pl.ds(j * 16, 16)] - half
            pos = (lax.iota(jnp.int32, 16) + j * 16) * 16
            for ci, tv in ((0, xt_v), (1, yt_v), (2, zt_v)):
                vals = plsc.load_gather(tv, [iv])
                plsc.store_scatter(xrows_v, [pos + ci], vals)
        pltpu.sync_copy(xrows_v, gxyzf_hbm.at[pl.ds(off * 16, _CH * 16)])

    pltpu.sync_copy(idx_hbm.at[pl.ds(base, _CH)], idx_a)
    pltpu.async_copy(feat_hbm.at[idx_a], frows_a, sem_a)

    def pair(c2, carry):
        off_e = base + (2 * c2) * _CH
        off_o = off_e + _CH
        pltpu.sync_copy(idx_hbm.at[pl.ds(off_o, _CH)], idx_b)
        pltpu.async_copy(feat_hbm.at[idx_b], frows_b, sem_b)
        xyz_block(idx_a, off_e)
        pltpu.make_async_copy(feat_hbm.at[idx_a], frows_a, sem_a).wait()
        pltpu.sync_copy(frows_a, gfeat_hbm.at[pl.ds(off_e, _CH)])

        @pl.when(c2 < _NCH // 2 - 1)
        def _():
            pltpu.sync_copy(idx_hbm.at[pl.ds(off_o + _CH, _CH)], idx_a)
            pltpu.async_copy(feat_hbm.at[idx_a], frows_a, sem_a)

        xyz_block(idx_b, off_o)
        pltpu.make_async_copy(feat_hbm.at[idx_b], frows_b, sem_b).wait()
        pltpu.sync_copy(frows_b, gfeat_hbm.at[pl.ds(off_o, _CH)])
        return carry

    lax.fori_loop(0, _NCH // 2, pair, 0)


@functools.lru_cache(maxsize=None)
def _make_gather():
    return pl.kernel(
        _gather_body,
        out_type=(
            jax.ShapeDtypeStruct((ROWS, CIN), jnp.float32),
            jax.ShapeDtypeStruct((ROWS * 16,), jnp.float32),
        ),
        mesh=plsc.VectorSubcoreMesh(core_axis_name="c", subcore_axis_name="s"),
        compiler_params=pltpu.CompilerParams(needs_layout_passes=False),
        scratch_types=[
            pltpu.VMEM((_CH,), jnp.int32),
            pltpu.VMEM((_CH,), jnp.int32),
            pltpu.VMEM((_CH, CIN), jnp.float32),
            pltpu.VMEM((_CH, CIN), jnp.float32),
            pltpu.VMEM((_CH * 16,), jnp.float32),
            pltpu.VMEM((4 * N,), jnp.float32),
            pltpu.VMEM((4 * N,), jnp.float32),
            pltpu.VMEM((4 * N,), jnp.float32),
            pltpu.SemaphoreType.DMA,
            pltpu.SemaphoreType.DMA,
        ],
    )


def _gather_call(feat_flat, xt, yt, zt, idx_flat):
    gfeat, gxyzf = _make_gather()(feat_flat, xt, yt, zt, idx_flat)
    return gfeat, gxyzf.reshape(ROWS, 16)


_RT = 2048


def _matmul_ext(gf, gx, qx, w_ref):
    y = jnp.dot(gf, w_ref[0:CIN, :], preferred_element_type=jnp.float32)
    y += jnp.dot(gx, w_ref[CIN:CIN + 16, :], preferred_element_type=jnp.float32)
    y += jnp.dot(qx, w_ref[CIN + 16:CIN + 32, :], preferred_element_type=jnp.float32)
    return y


def _mlp1_body(gf_ref, gx_ref, qx_ref, w_ref, s_ref, ss_ref):
    i = pl.program_id(0)
    y = _matmul_ext(gf_ref[...], gx_ref[...], qx_ref[...], w_ref)

    @pl.when(i == 0)
    def _():
        s_ref[...] = jnp.zeros_like(s_ref)
        ss_ref[...] = jnp.zeros_like(ss_ref)

    s_ref[...] += jnp.sum(y, axis=0, keepdims=True)
    ss_ref[...] += jnp.sum(y * y, axis=0, keepdims=True)


_GT = B * S // _RT


def _mlp1_call(gfeat, gxyz, qexp, wext):
    return pl.pallas_call(
        _mlp1_body,
        grid=(ROWS // _RT,),
        in_specs=[
            pl.BlockSpec((_RT, CIN), lambda i: (i, 0)),
            pl.BlockSpec((_RT, 16), lambda i: (i, 0)),
            pl.BlockSpec((_RT, 16), lambda i: (i % _GT, 0)),
            pl.BlockSpec((CIN + 32, COUT), lambda i: (0, 0)),
        ],
        out_specs=(
            pl.BlockSpec((1, COUT), lambda i: (0, 0)),
            pl.BlockSpec((1, COUT), lambda i: (0, 0)),
        ),
        out_shape=(
            jax.ShapeDtypeStruct((1, COUT), jnp.float32),
            jax.ShapeDtypeStruct((1, COUT), jnp.float32),
        ),
    )(gfeat, gxyz, qexp, wext)


def _mlp2_body(gf_ref, gx_ref, qx_ref, w_ref, w2_ref, sc_ref, sh_ref,
               s_ref, ss_ref):
    i = pl.program_id(0)
    y = _matmul_ext(gf_ref[...], gx_ref[...], qx_ref[...], w_ref)
    h = jnp.maximum(y * sc_ref[...] + sh_ref[...], 0.0)
    y2 = jnp.dot(h, w2_ref[...], preferred_element_type=jnp.float32)

    @pl.when(i == 0)
    def _():
        s_ref[...] = jnp.zeros_like(s_ref)
        ss_ref[...] = jnp.zeros_like(ss_ref)

    s_ref[...] += jnp.sum(y2, axis=0, keepdims=True)
    ss_ref[...] += jnp.sum(y2 * y2, axis=0, keepdims=True)


def _mlp2_call(gfeat, gxyz, qexp, wext, w2t, sc1, sh1):
    return pl.pallas_call(
        _mlp2_body,
        grid=(ROWS // _RT,),
        in_specs=[
            pl.BlockSpec((_RT, CIN), lambda i: (i, 0)),
            pl.BlockSpec((_RT, 16), lambda i: (i, 0)),
            pl.BlockSpec((_RT, 16), lambda i: (i % _GT, 0)),
            pl.BlockSpec((CIN + 32, COUT), lambda i: (0, 0)),
            pl.BlockSpec((COUT, COUT), lambda i: (0, 0)),
            pl.BlockSpec((1, COUT), lambda i: (0, 0)),
            pl.BlockSpec((1, COUT), lambda i: (0, 0)),
        ],
        out_specs=(
            pl.BlockSpec((1, COUT), lambda i: (0, 0)),
            pl.BlockSpec((1, COUT), lambda i: (0, 0)),
        ),
        out_shape=(
            jax.ShapeDtypeStruct((1, COUT), jnp.float32),
            jax.ShapeDtypeStruct((1, COUT), jnp.float32),
        ),
    )(gfeat, gxyz, qexp, wext, w2t, sc1, sh1)


def _mlp3_body(gf_ref, gx_ref, qx_ref, w_ref, w2_ref, sc1_ref, sh1_ref,
               sc2_ref, sh2_ref, o_ref):
    k = pl.program_id(1)
    y = _matmul_ext(gf_ref[...], gx_ref[...], qx_ref[...], w_ref)
    h = jnp.maximum(y * sc1_ref[...] + sh1_ref[...], 0.0)
    y2 = jnp.dot(h, w2_ref[...], preferred_element_type=jnp.float32)
    z = jnp.maximum(y2 * sc2_ref[...] + sh2_ref[...], 0.0)

    @pl.when(k == 0)
    def _():
        o_ref[...] = z

    @pl.when(k > 0)
    def _():
        o_ref[...] = jnp.maximum(o_ref[...], z)


def _mlp3_call(gfeat, gxyz, qexp, wext, w2t, sc1, sh1, sc2, sh2):
    return pl.pallas_call(
        _mlp3_body,
        grid=(_GT, KNN),
        in_specs=[
            pl.BlockSpec((_RT, CIN), lambda g, k: (k * _GT + g, 0)),
            pl.BlockSpec((_RT, 16), lambda g, k: (k * _GT + g, 0)),
            pl.BlockSpec((_RT, 16), lambda g, k: (g, 0)),
            pl.BlockSpec((CIN + 32, COUT), lambda g, k: (0, 0)),
            pl.BlockSpec((COUT, COUT), lambda g, k: (0, 0)),
            pl.BlockSpec((1, COUT), lambda g, k: (0, 0)),
            pl.BlockSpec((1, COUT), lambda g, k: (0, 0)),
            pl.BlockSpec((1, COUT), lambda g, k: (0, 0)),
            pl.BlockSpec((1, COUT), lambda g, k: (0, 0)),
        ],
        out_specs=pl.BlockSpec((_RT, COUT), lambda g, k: (g, 0)),
        out_shape=jax.ShapeDtypeStruct((B * S, COUT), jnp.float32),
    )(gfeat, gxyz, qexp, wext, w2t, sc1, sh1, sc2, sh2)


def kernel(xyz, feat, W1, g1, b1, W2, g2, b2):
    xyz3 = jnp.transpose(xyz, (0, 2, 1))
    far0 = jax.random.randint(jax.random.key(1), (B,), 0, N, dtype=jnp.int32)
    init = jnp.broadcast_to(far0[:, None], (B, 128))

    cxs, cys, czs = _fps_call(xyz3, init)
    new_xyz = jnp.stack([cxs, cys, czs], axis=-1)

    idx = _knn_call(xyz3, new_xyz)

    idx_km = jnp.transpose(idx, (2, 0, 1))
    idx_flat = (idx_km + (jnp.arange(B, dtype=jnp.int32) * N)[None, :, None])
    idx_flat = idx_flat.reshape(ROWS).astype(jnp.int32)

    feat_flat = feat.reshape(B * N, CIN)
    xt = xyz[:, :, 0].reshape(B * N)
    yt = xyz[:, :, 1].reshape(B * N)
    zt = xyz[:, :, 2].reshape(B * N)
    gfeat, gxyz = _gather_call(feat_flat, xt, yt, zt, idx_flat)

    qexp = jnp.pad(new_xyz, ((0, 0), (0, 0), (0, 13))).reshape(B * S, 16)

    w1t = W1.T
    zpad = jnp.zeros((13, COUT), dtype=jnp.float32)
    wext = jnp.concatenate(
        [w1t[:CIN], w1t[CIN:], zpad, -w1t[CIN:], zpad], axis=0)

    s1, ss1 = _mlp1_call(gfeat, gxyz, qexp, wext)
    n = jnp.float32(ROWS)
    m1 = s1 / n
    v1 = ss1 / n - m1 * m1
    sc1 = g1[None, :] * lax.rsqrt(v1 + 1e-5)
    sh1 = b1[None, :] - m1 * sc1

    s2, ss2 = _mlp2_call(gfeat, gxyz, qexp, wext, W2.T, sc1, sh1)
    m2 = s2 / n
    v2 = ss2 / n - m2 * m2
    sc2 = g2[None, :] * lax.rsqrt(v2 + 1e-5)
    sh2 = b2[None, :] - m2 * sc2

    nf = _mlp3_call(gfeat, gxyz, qexp, wext, W2.T, sc1, sh1, sc2, sh2)
    return new_xyz, nf.reshape(B, S, COUT)
